# baseline (speedup 1.0000x reference)
import jax
import jax.numpy as jnp
from jax.experimental import pallas as pl

_T = 32
_H = 32
_E = 32
_C = 128


def _lstm_kernel(xt_ref, lens_ref, embt_ref, wihf_ref, bf_ref, whhf_ref,
                 wihb_ref, bb_ref, whhb_ref, out_ref):
    f32 = jnp.float32
    embt = embt_ref[...]
    pft = jnp.dot(wihf_ref[...], embt, preferred_element_type=f32) + bf_ref[...]
    pbt = jnp.dot(wihb_ref[...], embt, preferred_element_type=f32) + bb_ref[...]
    pt = jnp.concatenate([pft, pbt], axis=0)

    xbt = xt_ref[...]
    bb_cols = xbt.shape[1]
    ids = jax.lax.broadcasted_iota(jnp.int32, (_C, bb_cols), 0)

    def gx(t, half):
        onehot_t = (xbt[t:t + 1, :] == ids).astype(f32)
        table = pt[4 * _H * half:4 * _H * (half + 1), :]
        return jnp.dot(table, onehot_t, preferred_element_type=f32)

    lens_b = lens_ref[...]
    whh_f = whhf_ref[...]
    whh_b = whhb_ref[...]

    def sig(v):
        return 0.5 * jnp.tanh(0.5 * v) + 0.5

    def step_dir(h, c, gin, t, whh):
        gates = gin + jnp.dot(whh, h, preferred_element_type=f32)
        i = sig(gates[:_H, :])
        f = sig(gates[_H:2 * _H, :])
        g = jnp.tanh(gates[2 * _H:3 * _H, :])
        o = sig(gates[3 * _H:, :])
        c_new = f * c + i * g
        h_new = o * jnp.tanh(c_new)
        m = t < lens_b
        return jnp.where(m, h_new, h), jnp.where(m, c_new, c)

    zeros = jnp.zeros((_H, bb_cols), f32)
    h_f, c_f, h_b, c_b = zeros, zeros, zeros, zeros
    for k in range(_T):
        tb = _T - 1 - k
        h_f, c_f = step_dir(h_f, c_f, gx(k, 0), k, whh_f)
        h_b, c_b = step_dir(h_b, c_b, gx(tb, 1), tb, whh_b)

    out_ref[...] = jnp.concatenate([h_f, h_b], axis=0).T


def kernel(x, lens, embed, Wih_f, Whh_f, bih_f, bhh_f, Wih_b, Whh_b, bih_b, bhh_b):
    B, T = x.shape
    assert T == _T
    BB = min(4096, B)
    grid = (B // BB,)

    xt = x.T
    lens2 = lens.reshape(1, B).astype(jnp.int32)
    bf = (bih_f + bhh_f).reshape(4 * _H, 1)
    bbias = (bih_b + bhh_b).reshape(4 * _H, 1)

    full = lambda shape: pl.BlockSpec(shape, lambda i: (0, 0))
    out = pl.pallas_call(
        _lstm_kernel,
        grid=grid,
        in_specs=[
            pl.BlockSpec((_T, BB), lambda i: (0, i)),
            pl.BlockSpec((1, BB), lambda i: (0, i)),
            full((_E, _C)),
            full((4 * _H, _E)),
            full((4 * _H, 1)),
            full((4 * _H, _H)),
            full((4 * _H, _E)),
            full((4 * _H, 1)),
            full((4 * _H, _H)),
        ],
        out_specs=pl.BlockSpec((BB, 2 * _H), lambda i: (i, 0)),
        out_shape=jax.ShapeDtypeStruct((B, 2 * _H), jnp.float32),
    )(xt, lens2, embed.T, Wih_f, bf, Whh_f, Wih_b, bbias, Whh_b)
    return out


# speedup vs baseline: 1.4065x; 1.4065x over previous
import jax
import jax.numpy as jnp
from jax.experimental import pallas as pl

_T = 32
_H = 32
_E = 32
_C = 128


def _lstm_kernel(xt_ref, lens_ref, embt_ref, wihf_ref, bf_ref, whhf_ref,
                 wihb_ref, bb_ref, whhb_ref, out_ref):
    f32 = jnp.float32
    embt = embt_ref[...]
    pft = jnp.dot(wihf_ref[...], embt, preferred_element_type=f32) + bf_ref[...]
    pbt = jnp.dot(wihb_ref[...], embt, preferred_element_type=f32) + bb_ref[...]
    pt = jnp.concatenate([pft, pbt], axis=0)

    xbt = xt_ref[...]
    bb_cols = xbt.shape[1]
    ids = jax.lax.broadcasted_iota(jnp.int32, (_C, bb_cols), 0)

    def gx(t, half):
        onehot_t = (xbt[t:t + 1, :] == ids).astype(f32)
        table = pt[4 * _H * half:4 * _H * (half + 1), :]
        return jnp.dot(table, onehot_t, preferred_element_type=f32)

    lens_b = lens_ref[...]
    whh_f = whhf_ref[...]
    whh_b = whhb_ref[...]

    def sig(v):
        return 0.5 * jnp.tanh(0.5 * v) + 0.5

    def step_dir(h, c, gin, t, whh):
        gates = gin + jnp.dot(whh, h, preferred_element_type=f32)
        i = sig(gates[:_H, :])
        f = sig(gates[_H:2 * _H, :])
        g = jnp.tanh(gates[2 * _H:3 * _H, :])
        o = sig(gates[3 * _H:, :])
        c_new = f * c + i * g
        h_new = o * jnp.tanh(c_new)
        m = t < lens_b
        return jnp.where(m, h_new, h), jnp.where(m, c_new, c)

    zeros = jnp.zeros((_H, bb_cols), f32)
    h_f, c_f, h_b, c_b = zeros, zeros, zeros, zeros
    for k in range(_T):
        tb = _T - 1 - k
        h_f, c_f = step_dir(h_f, c_f, gx(k, 0), k, whh_f)
        h_b, c_b = step_dir(h_b, c_b, gx(tb, 1), tb, whh_b)

    out_ref[...] = jnp.concatenate([h_f, h_b], axis=0).T


def kernel(x, lens, embed, Wih_f, Whh_f, bih_f, bhh_f, Wih_b, Whh_b, bih_b, bhh_b):
    B, T = x.shape
    assert T == _T
    BB = min(2048, B)
    grid = (B // BB,)

    xt = x.T
    lens2 = lens.reshape(1, B).astype(jnp.int32)
    bf = (bih_f + bhh_f).reshape(4 * _H, 1)
    bbias = (bih_b + bhh_b).reshape(4 * _H, 1)

    full = lambda shape: pl.BlockSpec(shape, lambda i: (0, 0))
    out = pl.pallas_call(
        _lstm_kernel,
        grid=grid,
        in_specs=[
            pl.BlockSpec((_T, BB), lambda i: (0, i)),
            pl.BlockSpec((1, BB), lambda i: (0, i)),
            full((_E, _C)),
            full((4 * _H, _E)),
            full((4 * _H, 1)),
            full((4 * _H, _H)),
            full((4 * _H, _E)),
            full((4 * _H, 1)),
            full((4 * _H, _H)),
        ],
        out_specs=pl.BlockSpec((BB, 2 * _H), lambda i: (i, 0)),
        out_shape=jax.ShapeDtypeStruct((B, 2 * _H), jnp.float32),
    )(xt, lens2, embed.T, Wih_f, bf, Whh_f, Wih_b, bbias, Whh_b)
    return out
